# baseline (speedup 1.0000x reference)
import functools

import jax
import jax.numpy as jnp
from jax import lax
from jax.experimental import pallas as pl
from jax.experimental.pallas import tpu as pltpu
from jax.experimental.pallas import tpu_sc as plsc

NUM_FEATURES = 155522
EMBED_DIM = 20
PAD_DIM = 32
NUM_USERS = 50000
BAG_LEN = 50
PBAG = 64
BATCH = 4096
NBAGS = 2 * BATCH


_BPW = NBAGS // 32


def _sc_body(epad_hbm, bowp_hbm, eli_hbm, out_hbm, uid_v, bow_v, rows_v, mean_v, sem):
    wid = lax.axis_index("s") * 2 + lax.axis_index("c")
    base = wid * _BPW
    pltpu.sync_copy(eli_hbm.at[pl.ds(base, _BPW)], uid_v)
    pltpu.async_copy(bowp_hbm.at[uid_v], bow_v, sem).wait()

    def bag(j, _):
        pltpu.async_copy(epad_hbm.at[bow_v.at[j]], rows_v, sem).wait()

        def body(p, accs):
            a0, a1, b0, b1 = accs
            a0 = a0 + rows_v[2 * p, pl.ds(0, 16)]
            a1 = a1 + rows_v[2 * p, pl.ds(16, 16)]
            b0 = b0 + rows_v[2 * p + 1, pl.ds(0, 16)]
            b1 = b1 + rows_v[2 * p + 1, pl.ds(16, 16)]
            return a0, a1, b0, b1

        z = jnp.zeros((16,), jnp.float32)
        a0, a1, b0, b1 = lax.fori_loop(0, PBAG // 2, body, (z, z, z, z))
        mean_v[j, pl.ds(0, 16)] = a0 + b0
        mean_v[j, pl.ds(16, 16)] = a1 + b1
        return 0

    lax.fori_loop(0, _BPW, bag, 0)
    pltpu.sync_copy(mean_v, out_hbm.at[pl.ds(base, _BPW)])


def _sc_bags(epad, bowp, eli_flat):
    mesh = plsc.VectorSubcoreMesh(core_axis_name="c", subcore_axis_name="s")
    f = functools.partial(
        pl.kernel,
        out_type=jax.ShapeDtypeStruct((NBAGS, PAD_DIM), jnp.float32),
        mesh=mesh,
        scratch_types=[
            pltpu.VMEM((_BPW,), jnp.int32),
            pltpu.VMEM((_BPW, PBAG), jnp.int32),
            pltpu.VMEM((PBAG, PAD_DIM), jnp.float32),
            pltpu.VMEM((_BPW, PAD_DIM), jnp.float32),
            pltpu.SemaphoreType.DMA,
        ],
        compiler_params=pltpu.CompilerParams(use_tc_tiling_on_sc=False),
    )(_sc_body)
    return f(epad, bowp, eli_flat)


def _mlp_body(m_ref, w1_ref, b1_ref, w2_ref, b2_ref, out_ref):
    m = m_ref[...]
    cnt = jnp.maximum(m[:, EMBED_DIM : EMBED_DIM + 1], 1.0)
    md = m / cnt
    x = jnp.concatenate([md[:BATCH], md[BATCH:]], axis=1)
    dn = (((1,), (1,)), ((), ()))
    h = jnp.maximum(lax.dot_general(x, w1_ref[...], dn) + b1_ref[...], 0.0)
    out_ref[...] = jnp.sum(h * w2_ref[...], axis=1, keepdims=True) + b2_ref[...]


def _mlp(means, w1p, b1, w2, b2):
    return pl.pallas_call(
        _mlp_body,
        out_shape=jax.ShapeDtypeStruct((BATCH, 1), jnp.float32),
    )(means, w1p, b1.reshape(1, 32), w2, b2.reshape(1, 1))


def kernel(BoW, edge_label_index, emb_weight, W1, b1, W2, b2):
    ss = jnp.sum(emb_weight * emb_weight, axis=1, keepdims=True)
    scale = jnp.minimum(1.0, 1.0 / jnp.maximum(jnp.sqrt(ss), 1e-7))
    ind = (jnp.arange(NUM_FEATURES, dtype=jnp.int32) != 0).astype(jnp.float32)
    ind = ind[:, None]
    epad = jnp.concatenate(
        [emb_weight * scale * ind, ind,
         jnp.zeros((NUM_FEATURES, PAD_DIM - EMBED_DIM - 1), jnp.float32)], axis=1)
    bowp = jnp.concatenate(
        [BoW, jnp.zeros((NUM_USERS, PBAG - BAG_LEN), jnp.int32)], axis=1)
    eli_flat = edge_label_index.reshape(NBAGS)

    means = _sc_bags(epad, bowp, eli_flat)

    zpad = jnp.zeros((32, PAD_DIM - EMBED_DIM), jnp.float32)
    w1p = jnp.concatenate([W1[:, :EMBED_DIM], zpad, W1[:, EMBED_DIM:], zpad], axis=1)
    out = _mlp(means, w1p, b1, W2, b2)
    return out.reshape(BATCH)


# speedup vs baseline: 1.0013x; 1.0013x over previous
import functools

import jax
import jax.numpy as jnp
from jax import lax
from jax.experimental import pallas as pl
from jax.experimental.pallas import tpu as pltpu
from jax.experimental.pallas import tpu_sc as plsc

NUM_FEATURES = 155522
EMBED_DIM = 20
PAD_DIM = 32
NUM_USERS = 50000
BAG_LEN = 50
PBAG = 64
BATCH = 4096
NBAGS = 2 * BATCH


_BPW = NBAGS // 32


_K = 8


def _sc_body(epad_hbm, bowp_hbm, eli_hbm, out_hbm, uid_v, bow_v, r0, r1, mean_v,
             s0, s1):
    wid = lax.axis_index("s") * 2 + lax.axis_index("c")
    base = wid * _BPW
    pltpu.sync_copy(eli_hbm.at[pl.ds(base, _BPW)], uid_v)
    pltpu.async_copy(bowp_hbm.at[uid_v], bow_v, s0).wait()

    def fire(g, rbuf, sem):
        for t in range(_K):
            pltpu.async_copy(
                epad_hbm.at[bow_v.at[g * _K + t]],
                rbuf.at[pl.ds(t * PBAG, PBAG)], sem)

    def drain(g, rbuf, sem):
        for t in range(_K):
            pltpu.make_async_copy(
                epad_hbm.at[bow_v.at[g * _K + t]],
                rbuf.at[pl.ds(t * PBAG, PBAG)], sem).wait()

    def accum(g, rbuf):
        def bag(t, _):
            def body(p, accs):
                a0, a1, b0, b1 = accs
                r = t * PBAG + 2 * p
                a0 = a0 + rbuf[r, pl.ds(0, 16)]
                a1 = a1 + rbuf[r, pl.ds(16, 16)]
                b0 = b0 + rbuf[r + 1, pl.ds(0, 16)]
                b1 = b1 + rbuf[r + 1, pl.ds(16, 16)]
                return a0, a1, b0, b1

            z = jnp.zeros((16,), jnp.float32)
            a0, a1, b0, b1 = lax.fori_loop(0, PBAG // 2, body, (z, z, z, z),
                                           unroll=8)
            mean_v[g * _K + t, pl.ds(0, 16)] = a0 + b0
            mean_v[g * _K + t, pl.ds(16, 16)] = a1 + b1
            return 0

        lax.fori_loop(0, _K, bag, 0)

    ngrp = _BPW // _K
    fire(0, r0, s0)

    def pair(i, _):
        ga = 2 * i
        fire(ga + 1, r1, s1)
        drain(ga, r0, s0)
        accum(ga, r0)

        @pl.when(ga + 2 < ngrp)
        def _():
            fire(ga + 2, r0, s0)

        drain(ga + 1, r1, s1)
        accum(ga + 1, r1)
        return 0

    lax.fori_loop(0, ngrp // 2, pair, 0)
    pltpu.sync_copy(mean_v, out_hbm.at[pl.ds(base, _BPW)])


def _sc_bags(epad, bowp, eli_flat):
    mesh = plsc.VectorSubcoreMesh(core_axis_name="c", subcore_axis_name="s")
    f = functools.partial(
        pl.kernel,
        out_type=jax.ShapeDtypeStruct((NBAGS, PAD_DIM), jnp.float32),
        mesh=mesh,
        scratch_types=[
            pltpu.VMEM((_BPW,), jnp.int32),
            pltpu.VMEM((_BPW, PBAG), jnp.int32),
            pltpu.VMEM((_K * PBAG, PAD_DIM), jnp.float32),
            pltpu.VMEM((_K * PBAG, PAD_DIM), jnp.float32),
            pltpu.VMEM((_BPW, PAD_DIM), jnp.float32),
            pltpu.SemaphoreType.DMA,
            pltpu.SemaphoreType.DMA,
        ],
        compiler_params=pltpu.CompilerParams(use_tc_tiling_on_sc=False),
    )(_sc_body)
    return f(epad, bowp, eli_flat)


def _mlp_body(m_ref, w1_ref, b1_ref, w2_ref, b2_ref, out_ref):
    m = m_ref[...]
    cnt = jnp.maximum(m[:, EMBED_DIM : EMBED_DIM + 1], 1.0)
    md = m / cnt
    x = jnp.concatenate([md[:BATCH], md[BATCH:]], axis=1)
    dn = (((1,), (1,)), ((), ()))
    h = jnp.maximum(lax.dot_general(x, w1_ref[...], dn) + b1_ref[...], 0.0)
    out_ref[...] = jnp.sum(h * w2_ref[...], axis=1, keepdims=True) + b2_ref[...]


def _mlp(means, w1p, b1, w2, b2):
    return pl.pallas_call(
        _mlp_body,
        out_shape=jax.ShapeDtypeStruct((BATCH, 1), jnp.float32),
    )(means, w1p, b1.reshape(1, 32), w2, b2.reshape(1, 1))


def kernel(BoW, edge_label_index, emb_weight, W1, b1, W2, b2):
    ss = jnp.sum(emb_weight * emb_weight, axis=1, keepdims=True)
    scale = jnp.minimum(1.0, 1.0 / jnp.maximum(jnp.sqrt(ss), 1e-7))
    ind = (jnp.arange(NUM_FEATURES, dtype=jnp.int32) != 0).astype(jnp.float32)
    ind = ind[:, None]
    epad = jnp.concatenate(
        [emb_weight * scale * ind, ind,
         jnp.zeros((NUM_FEATURES, PAD_DIM - EMBED_DIM - 1), jnp.float32)], axis=1)
    bowp = jnp.concatenate(
        [BoW, jnp.zeros((NUM_USERS, PBAG - BAG_LEN), jnp.int32)], axis=1)
    eli_flat = edge_label_index.reshape(NBAGS)

    means = _sc_bags(epad, bowp, eli_flat)

    zpad = jnp.zeros((32, PAD_DIM - EMBED_DIM), jnp.float32)
    w1p = jnp.concatenate([W1[:, :EMBED_DIM], zpad, W1[:, EMBED_DIM:], zpad], axis=1)
    out = _mlp(means, w1p, b1, W2, b2)
    return out.reshape(BATCH)


# speedup vs baseline: 1.0155x; 1.0142x over previous
import functools

import jax
import jax.numpy as jnp
from jax import lax
from jax.experimental import pallas as pl
from jax.experimental.pallas import tpu as pltpu
from jax.experimental.pallas import tpu_sc as plsc

NUM_FEATURES = 155522
EMBED_DIM = 20
PAD_DIM = 32
NUM_USERS = 50000
BAG_LEN = 50
PBAG = 64
BATCH = 4096
NBAGS = 2 * BATCH


_BPW = NBAGS // 32


_K = 16
_GROWS = _K * PBAG
_NGRP = _BPW // _K


def _sc_body(epad_hbm, bags_hbm, out_hbm, idx_v, r0, r1, mean_v, s0, s1):
    wid = lax.axis_index("s") * 2 + lax.axis_index("c")
    gbase = wid * _NGRP
    pltpu.sync_copy(bags_hbm.at[pl.ds(gbase, _NGRP)], idx_v)

    def fire(g, rbuf, sem):
        pltpu.async_copy(epad_hbm.at[idx_v.at[g]], rbuf, sem)

    def drain(g, rbuf, sem):
        pltpu.make_async_copy(epad_hbm.at[idx_v.at[g]], rbuf, sem).wait()

    def accum(g, rbuf):
        def bag(t, _):
            def body(p, accs):
                a0, a1, b0, b1 = accs
                r = t * PBAG + 2 * p
                a0 = a0 + rbuf[r, pl.ds(0, 16)]
                a1 = a1 + rbuf[r, pl.ds(16, 16)]
                b0 = b0 + rbuf[r + 1, pl.ds(0, 16)]
                b1 = b1 + rbuf[r + 1, pl.ds(16, 16)]
                return a0, a1, b0, b1

            z = jnp.zeros((16,), jnp.float32)
            a0, a1, b0, b1 = lax.fori_loop(0, PBAG // 2, body, (z, z, z, z),
                                           unroll=8)
            mean_v[g * _K + t, pl.ds(0, 16)] = a0 + b0
            mean_v[g * _K + t, pl.ds(16, 16)] = a1 + b1
            return 0

        lax.fori_loop(0, _K, bag, 0)

    fire(0, r0, s0)

    def pair(i, _):
        ga = 2 * i
        fire(ga + 1, r1, s1)
        drain(ga, r0, s0)
        accum(ga, r0)

        @pl.when(ga + 2 < _NGRP)
        def _():
            fire(ga + 2, r0, s0)

        drain(ga + 1, r1, s1)
        accum(ga + 1, r1)
        return 0

    lax.fori_loop(0, _NGRP // 2, pair, 0)
    pltpu.sync_copy(mean_v, out_hbm.at[pl.ds(wid * _BPW, _BPW)])


def _sc_bags(epad, bags):
    mesh = plsc.VectorSubcoreMesh(core_axis_name="c", subcore_axis_name="s")
    f = functools.partial(
        pl.kernel,
        out_type=jax.ShapeDtypeStruct((NBAGS, PAD_DIM), jnp.float32),
        mesh=mesh,
        scratch_types=[
            pltpu.VMEM((_NGRP, _GROWS), jnp.int32),
            pltpu.VMEM((_GROWS, PAD_DIM), jnp.float32),
            pltpu.VMEM((_GROWS, PAD_DIM), jnp.float32),
            pltpu.VMEM((_BPW, PAD_DIM), jnp.float32),
            pltpu.SemaphoreType.DMA,
            pltpu.SemaphoreType.DMA,
        ],
        compiler_params=pltpu.CompilerParams(use_tc_tiling_on_sc=False),
    )(_sc_body)
    return f(epad, bags)


def _mlp_body(m_ref, w1_ref, b1_ref, w2_ref, b2_ref, out_ref):
    m = m_ref[...]
    cnt = jnp.maximum(m[:, EMBED_DIM : EMBED_DIM + 1], 1.0)
    md = m / cnt
    x = jnp.concatenate([md[:BATCH], md[BATCH:]], axis=1)
    dn = (((1,), (1,)), ((), ()))
    h = jnp.maximum(lax.dot_general(x, w1_ref[...], dn) + b1_ref[...], 0.0)
    out_ref[...] = jnp.sum(h * w2_ref[...], axis=1, keepdims=True) + b2_ref[...]


def _mlp(means, w1p, b1, w2, b2):
    return pl.pallas_call(
        _mlp_body,
        out_shape=jax.ShapeDtypeStruct((BATCH, 1), jnp.float32),
    )(means, w1p, b1.reshape(1, 32), w2, b2.reshape(1, 1))


def kernel(BoW, edge_label_index, emb_weight, W1, b1, W2, b2):
    ss = jnp.sum(emb_weight * emb_weight, axis=1, keepdims=True)
    scale = jnp.minimum(1.0, 1.0 / jnp.maximum(jnp.sqrt(ss), 1e-7))
    ind = (jnp.arange(NUM_FEATURES, dtype=jnp.int32) != 0).astype(jnp.float32)
    ind = ind[:, None]
    epad = jnp.concatenate(
        [emb_weight * scale * ind, ind,
         jnp.zeros((NUM_FEATURES, PAD_DIM - EMBED_DIM - 1), jnp.float32)], axis=1)
    bowp = jnp.concatenate(
        [BoW, jnp.zeros((NUM_USERS, PBAG - BAG_LEN), jnp.int32)], axis=1)
    eli_flat = edge_label_index.reshape(NBAGS)
    bags = jnp.take(bowp, eli_flat, axis=0).reshape(32 * _NGRP, _GROWS)

    means = _sc_bags(epad, bags)

    zpad = jnp.zeros((32, PAD_DIM - EMBED_DIM), jnp.float32)
    w1p = jnp.concatenate([W1[:, :EMBED_DIM], zpad, W1[:, EMBED_DIM:], zpad], axis=1)
    out = _mlp(means, w1p, b1, W2, b2)
    return out.reshape(BATCH)
